# Initial kernel scaffold; baseline (speedup 1.0000x reference)
import jax
import jax.numpy as jnp
from jax.experimental import pallas as pl


def kernel(H, knn_indices, knn_weights, Wq, Wk, mix):
    raise NotImplementedError("write your pallas kernel here")


# speedup vs baseline: 2.8755x; 2.8755x over previous
import functools

import jax
import jax.numpy as jnp
from jax import lax
from jax.experimental import pallas as pl
from jax.experimental.pallas import tpu as pltpu
from jax.experimental.pallas import tpu_sc as plsc

_NC = 2
_NS = 16
_NW = _NC * _NS
_LANES = 16

_TAU = 0.2

_K = 32
_D = 128
_C = 4
_E = _C * _K


def _tc_qt_body(h_ref, wq_ref, wk_ref, qt_ref):
    m = lax.dot_general(wq_ref[...], wk_ref[...],
                        dimension_numbers=(((0,), (0,)), ((), ())),
                        preferred_element_type=jnp.float32,
                        precision=lax.Precision.HIGHEST)
    qt_ref[...] = jnp.dot(h_ref[...], m, preferred_element_type=jnp.float32,
                          precision=lax.Precision.HIGHEST)


def _make_sc_kernel(l_pad: int, n_chunks: int):
    mesh = plsc.VectorSubcoreMesh(core_axis_name="c", subcore_axis_name="s")
    rows_per_worker = l_pad // _NW
    chunks_per_worker = n_chunks // _NW

    @functools.partial(
        pl.kernel,
        out_type=jax.ShapeDtypeStruct((l_pad * _K,), jnp.float32),
        mesh=mesh,
        scratch_types=[
            pltpu.VMEM((_E,), jnp.int32),
            pltpu.VMEM((_E, _D), jnp.float32),
            pltpu.VMEM((_C * _D,), jnp.float32),
            pltpu.VMEM((_E,), jnp.float32),
            pltpu.VMEM((_LANES, _LANES), jnp.float32),
            pltpu.VMEM((_E,), jnp.float32),
            pltpu.VMEM((_LANES,), jnp.float32),
            pltpu.SemaphoreType.DMA,
        ],
        compiler_params=pltpu.CompilerParams(needs_layout_passes=False),
    )
    def sc_kernel(h_hbm, qt_hbm, idx_hbm, w_hbm, mix_hbm, out_hbm,
                  idx_v, g_v, qt_v, w_v, s_v, vals_v, mix_v, sem):
        wid = lax.axis_index("s") * _NC + lax.axis_index("c")
        base = wid * rows_per_worker

        pltpu.sync_copy(mix_hbm, mix_v)
        mixv = mix_v[...]
        beta = 1.0 / (1.0 + jnp.exp(-mixv))
        omb = 1.0 - beta

        iot = lax.iota(jnp.int32, _LANES)

        @pl.loop(0, chunks_per_worker)
        def _chunk(ci):
            r0 = base + ci * _C
            e0 = r0 * _K
            pltpu.sync_copy(idx_hbm.at[pl.ds(e0, _E)], idx_v)
            pltpu.async_copy(h_hbm.at[idx_v], g_v, sem).wait()
            pltpu.sync_copy(qt_hbm.at[pl.ds(r0 * _D, _C * _D)], qt_v)
            pltpu.sync_copy(w_hbm.at[pl.ds(e0, _E)], w_v)

            for r in range(_C):
                qt = [qt_v[pl.ds(r * _D + 16 * j, 16)] for j in range(8)]
                sims = []
                for g16 in range(2):
                    for t in range(_LANES):
                        e = r * _K + g16 * 16 + t
                        s = g_v[e, pl.ds(0, 16)] * qt[0]
                        for j in range(1, 8):
                            s = s + g_v[e, pl.ds(16 * j, 16)] * qt[j]
                        s_v[t, pl.ds(0, 16)] = s
                    tot = plsc.load_gather(s_v, [iot, jnp.zeros((16,), jnp.int32)])
                    for c in range(1, 16):
                        tot = tot + plsc.load_gather(
                            s_v, [iot, jnp.full((16,), c, jnp.int32)])
                    sims.append(tot)
                z0 = sims[0] * (1.0 / _TAU)
                z1 = sims[1] * (1.0 / _TAU)
                m = lax.reduce_max(jnp.maximum(z0, z1), axes=(0,))
                ex0 = jnp.exp(z0 - m)
                ex1 = jnp.exp(z1 - m)
                den = jnp.broadcast_to(lax.reduce_sum(ex0 + ex1, axes=(0,)), (16,))
                inv = 1.0 / den
                w0 = w_v[pl.ds(r * _K, 16)]
                w1 = w_v[pl.ds(r * _K + 16, 16)]
                wm0 = omb * w0 + beta * (ex0 * inv)
                wm1 = omb * w1 + beta * (ex1 * inv)
                den2 = jnp.broadcast_to(
                    lax.reduce_sum(wm0 + wm1, axes=(0,)), (16,)) + 1e-8
                inv2 = 1.0 / den2
                vals_v[pl.ds(r * _K, 16)] = wm0 * inv2
                vals_v[pl.ds(r * _K + 16, 16)] = wm1 * inv2

            pltpu.sync_copy(vals_v, out_hbm.at[pl.ds(e0, _E)])

    return sc_kernel


def kernel(H, knn_indices, knn_weights, Wq, Wk, mix):
    L, D = H.shape
    Lq, K = knn_indices.shape
    assert D == _D and K == _K and Lq == L

    qt = pl.pallas_call(
        _tc_qt_body,
        out_shape=jax.ShapeDtypeStruct((L, D), jnp.float32),
    )(H, Wq, Wk)

    rows_unit = _NW * _C
    l_pad = ((L + rows_unit - 1) // rows_unit) * rows_unit
    n_chunks = l_pad // _C
    pad = l_pad - L

    qt_flat = jnp.pad(qt, ((0, pad), (0, 0))).reshape(-1)
    idx_flat = jnp.pad(knn_indices, ((0, pad), (0, 0))).reshape(-1)
    w_flat = jnp.pad(knn_weights.astype(jnp.float32), ((0, pad), (0, 0))).reshape(-1)
    mix_vec = jnp.full((_LANES,), mix, dtype=jnp.float32)

    sc = _make_sc_kernel(l_pad, n_chunks)
    vals_pad = sc(H, qt_flat, idx_flat, w_flat, mix_vec)
    vals_flat = vals_pad[: L * K]

    rows_flat = jnp.broadcast_to(
        jnp.arange(L, dtype=jnp.int32)[:, None], (L, K)).reshape(-1)
    cols_flat = knn_indices.reshape(-1)
    return rows_flat, cols_flat, vals_flat
